# baseline (speedup 1.0000x reference)
import jax
import jax.numpy as jnp
from jax import lax
from jax.experimental import pallas as pl
from jax.experimental.pallas import tpu as pltpu
from jax.experimental.pallas import tpu_sc as plsc

EMB_DIM = 32
NUM_CORES = 2
NUM_SUBCORES = 16
NUM_WORKERS = NUM_CORES * NUM_SUBCORES
CHUNK_B = 32
VEC = 16


def _flatten_ids(token_ids):
    B, L = token_ids.shape
    b_per_worker = B // NUM_WORKERS
    n_flat = b_per_worker * L
    mesh = plsc.VectorSubcoreMesh(core_axis_name="core", subcore_axis_name="subcore")

    @pl.kernel(
        out_type=jax.ShapeDtypeStruct((B * L,), jnp.int32),
        mesh=mesh,
        scratch_types=[
            pltpu.VMEM((b_per_worker, L), jnp.int32),
            pltpu.VMEM((n_flat,), jnp.int32),
            pltpu.SemaphoreType.DMA,
        ],
        compiler_params=pltpu.CompilerParams(use_tc_tiling_on_sc=True),
    )
    def flatten_kernel(ids_hbm, out_hbm, ids_v, flat_v, sem):
        wid = lax.axis_index("subcore") * NUM_CORES + lax.axis_index("core")
        b0 = wid * b_per_worker
        pltpu.sync_copy(ids_hbm.at[pl.ds(b0, b_per_worker)], ids_v)

        @pl.loop(0, b_per_worker)
        def _(r):
            @pl.loop(0, L - VEC + 1, step=VEC)
            def _(c):
                flat_v[pl.ds(r * L + c, VEC)] = ids_v[r, pl.ds(c, VEC)]

            rem = L % VEC
            if rem:
                c_last = L - VEC
                flat_v[pl.ds(r * L + c_last, VEC)] = ids_v[r, pl.ds(c_last, VEC)]

        pltpu.sync_copy(flat_v, out_hbm.at[pl.ds(wid * n_flat, n_flat)])

    return flatten_kernel(token_ids)


def _depad_table(W):
    V, D = W.shape
    R = 400
    total_rounds = V // R
    full_rounds = total_rounds // NUM_WORKERS
    tail_rounds = total_rounds - full_rounds * NUM_WORKERS
    mesh = plsc.VectorSubcoreMesh(core_axis_name="core", subcore_axis_name="subcore")

    @pl.kernel(
        out_type=jax.ShapeDtypeStruct((V * D,), jnp.float32),
        mesh=mesh,
        scratch_types=[
            pltpu.VMEM((R, D), jnp.float32),
            pltpu.VMEM((R * D,), jnp.float32),
            pltpu.SemaphoreType.DMA,
        ],
        compiler_params=pltpu.CompilerParams(use_tc_tiling_on_sc=True),
    )
    def depad_kernel(w_hbm, out_hbm, tile_v, flat_v, sem):
        wid = lax.axis_index("subcore") * NUM_CORES + lax.axis_index("core")

        def one_round(rnd):
            r0 = rnd * R
            pltpu.sync_copy(w_hbm.at[pl.ds(r0, R)], tile_v)

            @pl.loop(0, R)
            def _(r):
                @pl.loop(0, D, step=VEC)
                def _(c):
                    flat_v[pl.ds(r * D + c, VEC)] = tile_v[r, pl.ds(c, VEC)]

            pltpu.sync_copy(flat_v, out_hbm.at[pl.ds(r0 * D, R * D)])

        @pl.loop(0, full_rounds)
        def _(t):
            one_round(t * NUM_WORKERS + wid)

        if tail_rounds:

            @pl.when(wid < tail_rounds)
            def _():
                one_round(full_rounds * NUM_WORKERS + wid)

    return depad_kernel(W)


def kernel(token_ids, W):
    B, L = token_ids.shape
    num_idx = B * L
    idx = _flatten_ids(token_ids.astype(jnp.int32))
    w_lin = _depad_table(W).reshape(W.shape[0], EMB_DIM)
    chunk = CHUNK_B * L
    b_per_worker = B // NUM_WORKERS
    n_chunks = b_per_worker // CHUNK_B

    mesh = plsc.VectorSubcoreMesh(core_axis_name="core", subcore_axis_name="subcore")

    @pl.kernel(
        out_type=jax.ShapeDtypeStruct((B, L, EMB_DIM), W.dtype),
        mesh=mesh,
        scratch_types=[
            pltpu.VMEM((chunk,), jnp.int32),
            pltpu.VMEM((chunk, EMB_DIM), jnp.float32),
            pltpu.SemaphoreType.DMA,
            pltpu.SemaphoreType.DMA,
        ],
        compiler_params=pltpu.CompilerParams(use_tc_tiling_on_sc=False),
    )
    def gather_kernel(w_hbm, i_hbm, o_hbm, idx_v, rows_v, gsem, wsem):
        wid = lax.axis_index("subcore") * NUM_CORES + lax.axis_index("core")
        wb = wid * b_per_worker

        @pl.loop(0, n_chunks)
        def _(c):
            b0 = wb + c * CHUNK_B
            pltpu.sync_copy(i_hbm.at[pl.ds(b0 * L, chunk)], idx_v)
            pltpu.async_copy(w_hbm.at[idx_v], rows_v, gsem).wait()

            @pl.loop(0, CHUNK_B)
            def _(j):
                pltpu.async_copy(rows_v.at[pl.ds(j * L, L)], o_hbm.at[b0 + j], wsem)

            @pl.loop(0, CHUNK_B)
            def _(j):
                pltpu.make_async_copy(
                    rows_v.at[pl.ds(j * L, L)], o_hbm.at[b0 + j], wsem
                ).wait()

    return gather_kernel(w_lin, idx)


# speedup vs baseline: 1.3541x; 1.3541x over previous
import jax
import jax.numpy as jnp
from jax import lax
from jax.experimental import pallas as pl
from jax.experimental.pallas import tpu as pltpu
from jax.experimental.pallas import tpu_sc as plsc

EMB_DIM = 32
NUM_CORES = 2
NUM_SUBCORES = 16
NUM_WORKERS = NUM_CORES * NUM_SUBCORES
CHUNK_B = 32
VEC = 16


def _flatten_ids(token_ids):
    B, L = token_ids.shape
    b_per_worker = B // NUM_WORKERS
    n_flat = b_per_worker * L
    mesh = plsc.VectorSubcoreMesh(core_axis_name="core", subcore_axis_name="subcore")

    @pl.kernel(
        out_type=jax.ShapeDtypeStruct((B * L,), jnp.int32),
        mesh=mesh,
        scratch_types=[
            pltpu.VMEM((b_per_worker, L), jnp.int32),
            pltpu.VMEM((n_flat,), jnp.int32),
            pltpu.SemaphoreType.DMA,
        ],
        compiler_params=pltpu.CompilerParams(use_tc_tiling_on_sc=True),
    )
    def flatten_kernel(ids_hbm, out_hbm, ids_v, flat_v, sem):
        wid = lax.axis_index("subcore") * NUM_CORES + lax.axis_index("core")
        b0 = wid * b_per_worker
        pltpu.sync_copy(ids_hbm.at[pl.ds(b0, b_per_worker)], ids_v)

        @pl.loop(0, b_per_worker)
        def _(r):
            @pl.loop(0, L - VEC + 1, step=VEC)
            def _(c):
                flat_v[pl.ds(r * L + c, VEC)] = ids_v[r, pl.ds(c, VEC)]

            rem = L % VEC
            if rem:
                c_last = L - VEC
                flat_v[pl.ds(r * L + c_last, VEC)] = ids_v[r, pl.ds(c_last, VEC)]

        pltpu.sync_copy(flat_v, out_hbm.at[pl.ds(wid * n_flat, n_flat)])

    return flatten_kernel(token_ids)


NUM_PARTS = 4


def kernel(token_ids, W):
    B, L = token_ids.shape
    idx = _flatten_ids(token_ids.astype(jnp.int32))
    chunk = CHUNK_B * L
    b_part = B // NUM_PARTS
    b_per_worker = b_part // NUM_WORKERS
    n_chunks = b_per_worker // CHUNK_B

    mesh = plsc.VectorSubcoreMesh(core_axis_name="core", subcore_axis_name="subcore")

    def make_part(part):
        @pl.kernel(
            out_type=jax.ShapeDtypeStruct((b_part, L, EMB_DIM), W.dtype),
            mesh=mesh,
            scratch_types=[
                pltpu.VMEM((chunk,), jnp.int32),
                pltpu.VMEM((chunk, EMB_DIM), jnp.float32),
                pltpu.SemaphoreType.DMA,
                pltpu.SemaphoreType.DMA,
            ],
            compiler_params=pltpu.CompilerParams(use_tc_tiling_on_sc=False),
        )
        def gather_kernel(w_hbm, i_hbm, o_hbm, idx_v, rows_v, gsem, wsem):
            wid = lax.axis_index("subcore") * NUM_CORES + lax.axis_index("core")
            wb = wid * b_per_worker

            @pl.loop(0, n_chunks)
            def _(c):
                b0 = wb + c * CHUNK_B
                gb = part * b_part + b0
                pltpu.sync_copy(i_hbm.at[pl.ds(gb * L, chunk)], idx_v)
                pltpu.async_copy(w_hbm.at[idx_v], rows_v, gsem).wait()

                @pl.loop(0, CHUNK_B)
                def _(j):
                    pltpu.async_copy(
                        rows_v.at[pl.ds(j * L, L)], o_hbm.at[b0 + j], wsem
                    )

                @pl.loop(0, CHUNK_B)
                def _(j):
                    pltpu.make_async_copy(
                        rows_v.at[pl.ds(j * L, L)], o_hbm.at[b0 + j], wsem
                    ).wait()

        return gather_kernel

    parts = [make_part(q)(W, idx) for q in range(NUM_PARTS)]
    return jnp.concatenate(parts, axis=0)


# speedup vs baseline: 1.3635x; 1.0069x over previous
import jax
import jax.numpy as jnp
from jax import lax
from jax.experimental import pallas as pl
from jax.experimental.pallas import tpu as pltpu
from jax.experimental.pallas import tpu_sc as plsc

EMB_DIM = 32
NUM_CORES = 2
NUM_SUBCORES = 16
NUM_WORKERS = NUM_CORES * NUM_SUBCORES
CHUNK_B = 32
VEC = 16


def _flatten_ids(token_ids):
    B, L = token_ids.shape
    b_per_worker = B // NUM_WORKERS
    n_flat = b_per_worker * L
    mesh = plsc.VectorSubcoreMesh(core_axis_name="core", subcore_axis_name="subcore")

    @pl.kernel(
        out_type=jax.ShapeDtypeStruct((B * L,), jnp.int32),
        mesh=mesh,
        scratch_types=[
            pltpu.VMEM((b_per_worker, L), jnp.int32),
            pltpu.VMEM((n_flat,), jnp.int32),
            pltpu.SemaphoreType.DMA,
        ],
        compiler_params=pltpu.CompilerParams(use_tc_tiling_on_sc=True),
    )
    def flatten_kernel(ids_hbm, out_hbm, ids_v, flat_v, sem):
        wid = lax.axis_index("subcore") * NUM_CORES + lax.axis_index("core")
        b0 = wid * b_per_worker
        pltpu.sync_copy(ids_hbm.at[pl.ds(b0, b_per_worker)], ids_v)

        @pl.loop(0, b_per_worker)
        def _(r):
            @pl.loop(0, L - VEC + 1, step=VEC)
            def _(c):
                flat_v[pl.ds(r * L + c, VEC)] = ids_v[r, pl.ds(c, VEC)]

            rem = L % VEC
            if rem:
                c_last = L - VEC
                flat_v[pl.ds(r * L + c_last, VEC)] = ids_v[r, pl.ds(c_last, VEC)]

        pltpu.sync_copy(flat_v, out_hbm.at[pl.ds(wid * n_flat, n_flat)])

    return flatten_kernel(token_ids)


NUM_PARTS = 8


def kernel(token_ids, W):
    B, L = token_ids.shape
    idx = _flatten_ids(token_ids.astype(jnp.int32))
    chunk = CHUNK_B * L
    b_part = B // NUM_PARTS
    b_per_worker = b_part // NUM_WORKERS
    n_chunks = b_per_worker // CHUNK_B

    mesh = plsc.VectorSubcoreMesh(core_axis_name="core", subcore_axis_name="subcore")

    def make_part(part):
        @pl.kernel(
            out_type=jax.ShapeDtypeStruct((b_part, L, EMB_DIM), W.dtype),
            mesh=mesh,
            scratch_types=[
                pltpu.VMEM((chunk,), jnp.int32),
                pltpu.VMEM((chunk, EMB_DIM), jnp.float32),
                pltpu.SemaphoreType.DMA,
                pltpu.SemaphoreType.DMA,
            ],
            compiler_params=pltpu.CompilerParams(use_tc_tiling_on_sc=False),
        )
        def gather_kernel(w_hbm, i_hbm, o_hbm, idx_v, rows_v, gsem, wsem):
            wid = lax.axis_index("subcore") * NUM_CORES + lax.axis_index("core")
            wb = wid * b_per_worker

            @pl.loop(0, n_chunks)
            def _(c):
                b0 = wb + c * CHUNK_B
                gb = part * b_part + b0
                pltpu.sync_copy(i_hbm.at[pl.ds(gb * L, chunk)], idx_v)
                pltpu.async_copy(w_hbm.at[idx_v], rows_v, gsem).wait()

                @pl.loop(0, CHUNK_B)
                def _(j):
                    pltpu.async_copy(
                        rows_v.at[pl.ds(j * L, L)], o_hbm.at[b0 + j], wsem
                    )

                @pl.loop(0, CHUNK_B)
                def _(j):
                    pltpu.make_async_copy(
                        rows_v.at[pl.ds(j * L, L)], o_hbm.at[b0 + j], wsem
                    ).wait()

        return gather_kernel

    parts = [make_part(q)(W, idx) for q in range(NUM_PARTS)]
    return jnp.concatenate(parts, axis=0)


# speedup vs baseline: 1.3673x; 1.0028x over previous
import jax
import jax.numpy as jnp
from jax import lax
from jax.experimental import pallas as pl
from jax.experimental.pallas import tpu as pltpu
from jax.experimental.pallas import tpu_sc as plsc

EMB_DIM = 32
NUM_CORES = 2
NUM_SUBCORES = 16
NUM_WORKERS = NUM_CORES * NUM_SUBCORES
CHUNK_B = 64
VEC = 16


def _flatten_ids(token_ids):
    B, L = token_ids.shape
    b_per_worker = B // NUM_WORKERS
    n_flat = b_per_worker * L
    mesh = plsc.VectorSubcoreMesh(core_axis_name="core", subcore_axis_name="subcore")

    @pl.kernel(
        out_type=jax.ShapeDtypeStruct((B * L,), jnp.int32),
        mesh=mesh,
        scratch_types=[
            pltpu.VMEM((b_per_worker, L), jnp.int32),
            pltpu.VMEM((n_flat,), jnp.int32),
            pltpu.SemaphoreType.DMA,
        ],
        compiler_params=pltpu.CompilerParams(use_tc_tiling_on_sc=True),
    )
    def flatten_kernel(ids_hbm, out_hbm, ids_v, flat_v, sem):
        wid = lax.axis_index("subcore") * NUM_CORES + lax.axis_index("core")
        b0 = wid * b_per_worker
        pltpu.sync_copy(ids_hbm.at[pl.ds(b0, b_per_worker)], ids_v)

        @pl.loop(0, b_per_worker)
        def _(r):
            @pl.loop(0, L - VEC + 1, step=VEC)
            def _(c):
                flat_v[pl.ds(r * L + c, VEC)] = ids_v[r, pl.ds(c, VEC)]

            rem = L % VEC
            if rem:
                c_last = L - VEC
                flat_v[pl.ds(r * L + c_last, VEC)] = ids_v[r, pl.ds(c_last, VEC)]

        pltpu.sync_copy(flat_v, out_hbm.at[pl.ds(wid * n_flat, n_flat)])

    return flatten_kernel(token_ids)


NUM_PARTS = 8


def kernel(token_ids, W):
    B, L = token_ids.shape
    idx = _flatten_ids(token_ids.astype(jnp.int32))
    chunk = CHUNK_B * L
    b_part = B // NUM_PARTS
    b_per_worker = b_part // NUM_WORKERS
    n_chunks = b_per_worker // CHUNK_B

    mesh = plsc.VectorSubcoreMesh(core_axis_name="core", subcore_axis_name="subcore")

    def make_part(part):
        @pl.kernel(
            out_type=jax.ShapeDtypeStruct((b_part, L, EMB_DIM), W.dtype),
            mesh=mesh,
            scratch_types=[
                pltpu.VMEM((chunk,), jnp.int32),
                pltpu.VMEM((chunk, EMB_DIM), jnp.float32),
                pltpu.SemaphoreType.DMA,
                pltpu.SemaphoreType.DMA,
            ],
            compiler_params=pltpu.CompilerParams(use_tc_tiling_on_sc=False),
        )
        def gather_kernel(w_hbm, i_hbm, o_hbm, idx_v, rows_v, gsem, wsem):
            wid = lax.axis_index("subcore") * NUM_CORES + lax.axis_index("core")
            wb = wid * b_per_worker

            @pl.loop(0, n_chunks)
            def _(c):
                b0 = wb + c * CHUNK_B
                gb = part * b_part + b0
                pltpu.sync_copy(i_hbm.at[pl.ds(gb * L, chunk)], idx_v)
                pltpu.async_copy(w_hbm.at[idx_v], rows_v, gsem).wait()

                @pl.loop(0, CHUNK_B)
                def _(j):
                    pltpu.async_copy(
                        rows_v.at[pl.ds(j * L, L)], o_hbm.at[b0 + j], wsem
                    )

                @pl.loop(0, CHUNK_B)
                def _(j):
                    pltpu.make_async_copy(
                        rows_v.at[pl.ds(j * L, L)], o_hbm.at[b0 + j], wsem
                    ).wait()

        return gather_kernel

    parts = [make_part(q)(W, idx) for q in range(NUM_PARTS)]
    return jnp.concatenate(parts, axis=0)
